# baseline (speedup 1.0000x reference)
import functools

import jax
import jax.numpy as jnp
from jax import lax
from jax.experimental import pallas as pl
from jax.experimental.pallas import tpu as pltpu
from jax.experimental.pallas import tpu_sc as plsc

_NEG = -1e9
_ROW_BLK = 512


def _argmax_body(k_ref, fcolt_ref, out_ref, kn_ref):
    i = pl.program_id(0)

    @pl.when(i == 0)
    def _():
        k = k_ref[...]
        norms = jnp.sqrt(jnp.sum(k * k, axis=0, keepdims=True)) + 1e-8
        kn_ref[...] = k / norms

    kn = kn_ref[...]
    q = k_ref[:, pl.ds(i * _ROW_BLK, _ROW_BLK)]
    simt = jax.lax.dot_general(
        kn, q, (((0,), (0,)), ((), ())),
        preferred_element_type=jnp.float32)
    fcolt = fcolt_ref[...]
    simt = jnp.where(fcolt >= 1, _NEG, simt)
    idx = jnp.argmax(simt, axis=0)
    out_ref[0] = idx.astype(jnp.int32)[None, :]


def _compute_idx(latter2d, fcolt):
    c2, n = latter2d.shape
    nblk = n // _ROW_BLK
    grid_spec = pl.GridSpec(
        grid=(nblk,),
        in_specs=[
            pl.BlockSpec((c2, n), lambda i: (0, 0)),
            pl.BlockSpec((n, 1), lambda i: (0, 0)),
        ],
        out_specs=pl.BlockSpec((1, 1, _ROW_BLK), lambda i: (i, 0, 0)),
        scratch_shapes=[pltpu.VMEM((c2, n), jnp.float32)],
    )
    out = pl.pallas_call(
        _argmax_body,
        grid_spec=grid_spec,
        out_shape=jax.ShapeDtypeStruct((nblk, 1, _ROW_BLK), jnp.int32),
    )(latter2d, fcolt)
    return out.reshape(n)


def _sc_shift(input4d, idx, flagf):
    _, c, h, w = input4d.shape
    c2 = c // 2
    n = h * w
    info = plsc.get_sparse_core_info()
    nc, ns = info.num_cores, info.num_subcores
    nw = nc * ns
    f_per_w = c2 // nw
    mesh = plsc.VectorSubcoreMesh(core_axis_name="c", subcore_axis_name="s")

    @functools.partial(
        pl.kernel, mesh=mesh,
        out_type=jax.ShapeDtypeStruct((1, c2, h, w), jnp.float32),
        scratch_types=[
            pltpu.VMEM((n,), jnp.int32),
            pltpu.VMEM((n,), jnp.float32),
            pltpu.VMEM((h, w), jnp.float32),
            pltpu.VMEM((h, w), jnp.float32),
        ],
        compiler_params=pltpu.CompilerParams(needs_layout_passes=False),
    )
    def shift_k(in_hbm, idx_hbm, flag_hbm, out_hbm, idx_v, flag_v, src_v,
                dst_v):
        wid = lax.axis_index("s") * nc + lax.axis_index("c")
        pltpu.sync_copy(idx_hbm, idx_v)
        pltpu.sync_copy(flag_hbm, flag_v)
        for j in range(f_per_w):
            f = wid + nw * j
            pltpu.sync_copy(in_hbm.at[0, f], src_v)

            log2w = w.bit_length() - 1
            chunks_per_row = w // 16

            @plsc.parallel_loop(0, n // 16, unroll=16)
            def _(t):
                vid = idx_v[pl.ds(t * 16, 16)]
                hi = lax.shift_right_logical(vid, log2w)
                lo = lax.bitwise_and(vid, jnp.int32(w - 1))
                g = plsc.load_gather(src_v, [hi, lo])
                fl = flag_v[pl.ds(t * 16, 16)]
                r = lax.shift_right_logical(t, 2)
                cc = lax.bitwise_and(t, chunks_per_row - 1) * 16
                dst_v[r, pl.ds(cc, 16)] = g * fl
            pltpu.sync_copy(dst_v, out_hbm.at[0, f])

    return shift_k(input4d, idx, flagf)


def kernel(input, mask):
    b, c, h, w = input.shape
    c2 = c // 2
    n = h * w
    latter2d = input.reshape(c, n)[c2:]
    flag = mask.reshape(n) >= 1
    fcolt = flag.reshape(n, 1).astype(jnp.int32)
    flagf = flag.astype(jnp.float32)

    idx = _compute_idx(latter2d, fcolt)
    shift_map = _sc_shift(input, idx, flagf)

    out0 = jnp.pad(input, ((0, 0), (0, c2), (0, 0), (0, 0)))
    return lax.dynamic_update_slice(out0, shift_map, (0, c, 0, 0))


# speedup vs baseline: 1.0575x; 1.0575x over previous
import functools

import jax
import jax.numpy as jnp
from jax import lax
from jax.experimental import pallas as pl
from jax.experimental.pallas import tpu as pltpu
from jax.experimental.pallas import tpu_sc as plsc

_NEG = -1e9
_ROW_BLK = 1024


def _argmax_body(k_ref, fcolt_ref, out_ref, kn_ref):
    i = pl.program_id(0)

    @pl.when(i == 0)
    def _():
        k = k_ref[...]
        norms = jnp.sqrt(jnp.sum(k * k, axis=0, keepdims=True)) + 1e-8
        kn_ref[...] = k / norms

    kn = kn_ref[...]
    q = k_ref[:, pl.ds(i * _ROW_BLK, _ROW_BLK)]
    simt = jax.lax.dot_general(
        kn, q, (((0,), (0,)), ((), ())),
        preferred_element_type=jnp.float32)
    fcolt = fcolt_ref[...]
    simt = jnp.where(fcolt >= 1, _NEG, simt)
    idx = jnp.argmax(simt, axis=0)
    out_ref[0] = idx.astype(jnp.int32)[None, :]


def _compute_idx(latter2d, fcolt):
    c2, n = latter2d.shape
    nblk = n // _ROW_BLK
    grid_spec = pl.GridSpec(
        grid=(nblk,),
        in_specs=[
            pl.BlockSpec((c2, n), lambda i: (0, 0)),
            pl.BlockSpec((n, 1), lambda i: (0, 0)),
        ],
        out_specs=pl.BlockSpec((1, 1, _ROW_BLK), lambda i: (i, 0, 0)),
        scratch_shapes=[pltpu.VMEM((c2, n), jnp.float32)],
    )
    out = pl.pallas_call(
        _argmax_body,
        grid_spec=grid_spec,
        out_shape=jax.ShapeDtypeStruct((nblk, 1, _ROW_BLK), jnp.int32),
    )(latter2d, fcolt)
    return out.reshape(n)


def _sc_shift(input4d, idx, flagf):
    _, c, h, w = input4d.shape
    c2 = c // 2
    n = h * w
    info = plsc.get_sparse_core_info()
    nc, ns = info.num_cores, info.num_subcores
    nw = nc * ns
    f_per_w = c2 // nw
    mesh = plsc.VectorSubcoreMesh(core_axis_name="c", subcore_axis_name="s")

    @functools.partial(
        pl.kernel, mesh=mesh,
        out_type=jax.ShapeDtypeStruct((1, c2, h, w), jnp.float32),
        scratch_types=[
            pltpu.VMEM((n,), jnp.int32),
            pltpu.VMEM((n,), jnp.float32),
            pltpu.VMEM((h, w), jnp.float32),
            pltpu.VMEM((h, w), jnp.float32),
        ],
        compiler_params=pltpu.CompilerParams(needs_layout_passes=False),
    )
    def shift_k(in_hbm, idx_hbm, flag_hbm, out_hbm, idx_v, flag_v, src_v,
                dst_v):
        wid = lax.axis_index("s") * nc + lax.axis_index("c")
        pltpu.sync_copy(idx_hbm, idx_v)
        pltpu.sync_copy(flag_hbm, flag_v)
        for j in range(f_per_w):
            f = wid + nw * j
            pltpu.sync_copy(in_hbm.at[0, f], src_v)

            log2w = w.bit_length() - 1
            chunks_per_row = w // 16

            @plsc.parallel_loop(0, n // 16, unroll=16)
            def _(t):
                vid = idx_v[pl.ds(t * 16, 16)]
                hi = lax.shift_right_logical(vid, log2w)
                lo = lax.bitwise_and(vid, jnp.int32(w - 1))
                g = plsc.load_gather(src_v, [hi, lo])
                fl = flag_v[pl.ds(t * 16, 16)]
                r = lax.shift_right_logical(t, 2)
                cc = lax.bitwise_and(t, chunks_per_row - 1) * 16
                dst_v[r, pl.ds(cc, 16)] = g * fl
            pltpu.sync_copy(dst_v, out_hbm.at[0, f])

    return shift_k(input4d, idx, flagf)


def kernel(input, mask):
    b, c, h, w = input.shape
    c2 = c // 2
    n = h * w
    latter2d = input.reshape(c, n)[c2:]
    flag = mask.reshape(n) >= 1
    fcolt = flag.reshape(n, 1).astype(jnp.int32)
    flagf = flag.astype(jnp.float32)

    idx = _compute_idx(latter2d, fcolt)
    shift_map = _sc_shift(input, idx, flagf)

    out0 = jnp.pad(input, ((0, 0), (0, c2), (0, 0), (0, 0)))
    return lax.dynamic_update_slice(out0, shift_map, (0, c, 0, 0))


# speedup vs baseline: 1.0956x; 1.0361x over previous
import functools

import jax
import jax.numpy as jnp
from jax import lax
from jax.experimental import pallas as pl
from jax.experimental.pallas import tpu as pltpu
from jax.experimental.pallas import tpu_sc as plsc

_NEG = -1e9
_ROW_BLK = 2048


def _argmax_body(k_ref, fcolt_ref, out_ref, kn_ref):
    i = pl.program_id(0)

    @pl.when(i == 0)
    def _():
        k = k_ref[...]
        norms = jnp.sqrt(jnp.sum(k * k, axis=0, keepdims=True)) + 1e-8
        kn_ref[...] = k / norms

    kn = kn_ref[...]
    q = k_ref[:, pl.ds(i * _ROW_BLK, _ROW_BLK)]
    simt = jax.lax.dot_general(
        kn, q, (((0,), (0,)), ((), ())),
        preferred_element_type=jnp.float32)
    fcolt = fcolt_ref[...]
    simt = jnp.where(fcolt >= 1, _NEG, simt)
    idx = jnp.argmax(simt, axis=0)
    out_ref[0] = idx.astype(jnp.int32)[None, :]


def _compute_idx(latter2d, fcolt):
    c2, n = latter2d.shape
    nblk = n // _ROW_BLK
    grid_spec = pl.GridSpec(
        grid=(nblk,),
        in_specs=[
            pl.BlockSpec((c2, n), lambda i: (0, 0)),
            pl.BlockSpec((n, 1), lambda i: (0, 0)),
        ],
        out_specs=pl.BlockSpec((1, 1, _ROW_BLK), lambda i: (i, 0, 0)),
        scratch_shapes=[pltpu.VMEM((c2, n), jnp.float32)],
    )
    out = pl.pallas_call(
        _argmax_body,
        grid_spec=grid_spec,
        out_shape=jax.ShapeDtypeStruct((nblk, 1, _ROW_BLK), jnp.int32),
    )(latter2d, fcolt)
    return out.reshape(n)


def _sc_shift(input4d, idx, flagf):
    _, c, h, w = input4d.shape
    c2 = c // 2
    n = h * w
    info = plsc.get_sparse_core_info()
    nc, ns = info.num_cores, info.num_subcores
    nw = nc * ns
    f_per_w = c2 // nw
    mesh = plsc.VectorSubcoreMesh(core_axis_name="c", subcore_axis_name="s")

    @functools.partial(
        pl.kernel, mesh=mesh,
        out_type=jax.ShapeDtypeStruct((1, c2, h, w), jnp.float32),
        scratch_types=[
            pltpu.VMEM((n,), jnp.int32),
            pltpu.VMEM((n,), jnp.float32),
            pltpu.VMEM((h, w), jnp.float32),
            pltpu.VMEM((h, w), jnp.float32),
        ],
        compiler_params=pltpu.CompilerParams(needs_layout_passes=False),
    )
    def shift_k(in_hbm, idx_hbm, flag_hbm, out_hbm, idx_v, flag_v, src_v,
                dst_v):
        wid = lax.axis_index("s") * nc + lax.axis_index("c")
        pltpu.sync_copy(idx_hbm, idx_v)
        pltpu.sync_copy(flag_hbm, flag_v)
        for j in range(f_per_w):
            f = wid + nw * j
            pltpu.sync_copy(in_hbm.at[0, f], src_v)

            log2w = w.bit_length() - 1
            chunks_per_row = w // 16

            @plsc.parallel_loop(0, n // 16, unroll=16)
            def _(t):
                vid = idx_v[pl.ds(t * 16, 16)]
                hi = lax.shift_right_logical(vid, log2w)
                lo = lax.bitwise_and(vid, jnp.int32(w - 1))
                g = plsc.load_gather(src_v, [hi, lo])
                fl = flag_v[pl.ds(t * 16, 16)]
                r = lax.shift_right_logical(t, 2)
                cc = lax.bitwise_and(t, chunks_per_row - 1) * 16
                dst_v[r, pl.ds(cc, 16)] = g * fl
            pltpu.sync_copy(dst_v, out_hbm.at[0, f])

    return shift_k(input4d, idx, flagf)


def kernel(input, mask):
    b, c, h, w = input.shape
    c2 = c // 2
    n = h * w
    latter2d = input.reshape(c, n)[c2:]
    flag = mask.reshape(n) >= 1
    fcolt = flag.reshape(n, 1).astype(jnp.int32)
    flagf = flag.astype(jnp.float32)

    idx = _compute_idx(latter2d, fcolt)
    shift_map = _sc_shift(input, idx, flagf)

    out0 = jnp.pad(input, ((0, 0), (0, c2), (0, 0), (0, 0)))
    return lax.dynamic_update_slice(out0, shift_map, (0, c, 0, 0))
